# Initial kernel scaffold; baseline (speedup 1.0000x reference)
import jax
import jax.numpy as jnp
from jax.experimental import pallas as pl


def kernel(weights, input):
    raise NotImplementedError("write your pallas kernel here")


# speedup vs baseline: 1.1602x; 1.1602x over previous
import functools
import math

import jax
import jax.numpy as jnp
from jax import lax
from jax.experimental import pallas as pl
from jax.experimental.pallas import tpu as pltpu
from jax.experimental.pallas import tpu_sc as plsc

_PAD = 1

_NC = 2
_NS = 16
_L = 16
_NW = _NC * _NS


def _positions_body(inp_ref, pos_ref, maskf_ref):
    x = inp_ref[...]
    bsz, seq = x.shape
    m = jnp.where(x != _PAD, 1, 0).astype(jnp.int32)
    c = m
    k = 1
    while k < seq:
        z = jnp.zeros((bsz, k), jnp.int32)
        c = c + jnp.concatenate([z, c[:, : seq - k]], axis=1)
        k *= 2
    pos_ref[...] = (c - 1) * m
    maskf_ref[...] = m.astype(jnp.float32)


def _compute_positions(inp):
    bsz, seq = inp.shape
    return pl.pallas_call(
        _positions_body,
        out_shape=(
            jax.ShapeDtypeStruct((bsz, seq), jnp.int32),
            jax.ShapeDtypeStruct((bsz, seq), jnp.float32),
        ),
    )(inp)


def _make_sc_gather(num_tokens, d_model, chunk):
    nchunks_total = num_tokens // chunk
    chunks_per_w = nchunks_total // _NW
    mesh = plsc.VectorSubcoreMesh(
        core_axis_name="c", subcore_axis_name="s", num_cores=_NC, num_subcores=_NS
    )

    @functools.partial(
        pl.kernel,
        mesh=mesh,
        compiler_params=pltpu.CompilerParams(needs_layout_passes=False),
        out_type=jax.ShapeDtypeStruct((num_tokens, d_model), jnp.float32),
        scratch_types=[
            pltpu.VMEM((chunks_per_w, chunk), jnp.int32),
            pltpu.VMEM((chunks_per_w, chunk), jnp.float32),
            pltpu.VMEM((chunk, d_model), jnp.float32),
            pltpu.SemaphoreType.DMA,
        ],
    )
    def sc_gather(table_hbm, pos_hbm, maskf_hbm, out_hbm, idx_v, mf_v, rows_v, sem):
        wid = lax.axis_index("s") * _NC + lax.axis_index("c")
        rbase = wid * chunks_per_w
        pltpu.sync_copy(pos_hbm.at[pl.ds(rbase, chunks_per_w)], idx_v)
        pltpu.sync_copy(maskf_hbm.at[pl.ds(rbase, chunks_per_w)], mf_v)
        for c in range(chunks_per_w):
            pltpu.async_copy(table_hbm.at[idx_v.at[c]], rows_v, sem).wait()

            def tok_body(t, carry):
                mvec = plsc.load_gather(
                    mf_v,
                    [jnp.full((_L,), c, jnp.int32), jnp.full((_L,), t, jnp.int32)],
                )
                for d in range(d_model // _L):
                    sl = (t, pl.ds(d * _L, _L))
                    rows_v[sl] = rows_v[sl] * mvec
                return carry

            lax.fori_loop(0, chunk, tok_body, 0)

            pltpu.sync_copy(
                rows_v, out_hbm.at[pl.ds(rbase * chunk + c * chunk, chunk)]
            )

    return sc_gather


def kernel(weights, input):
    bsz, seq = input.shape
    num_tokens = bsz * seq
    d_model = weights.shape[1]
    chunk = 32

    pos, maskf = _compute_positions(input.astype(jnp.int32))
    pos = pos.reshape(num_tokens // chunk, chunk)
    maskf = maskf.reshape(num_tokens // chunk, chunk)

    gather = _make_sc_gather(num_tokens, d_model, chunk)
    out = gather(weights, pos, maskf)
    return out.reshape(bsz, seq, d_model)


# speedup vs baseline: 1.6433x; 1.4164x over previous
import functools
import math

import jax
import jax.numpy as jnp
from jax import lax
from jax.experimental import pallas as pl
from jax.experimental.pallas import tpu as pltpu
from jax.experimental.pallas import tpu_sc as plsc

_PAD = 1

_NC = 2
_NS = 16
_L = 16
_NW = _NC * _NS


def _positions_body(inp_ref, pos_ref, maskf_ref):
    x = inp_ref[...]
    bsz, seq = x.shape
    m = jnp.where(x != _PAD, 1, 0).astype(jnp.int32)
    c = m
    k = 1
    while k < seq:
        z = jnp.zeros((bsz, k), jnp.int32)
        c = c + jnp.concatenate([z, c[:, : seq - k]], axis=1)
        k *= 2
    pos_ref[...] = (c - 1) * m
    maskf_ref[...] = m.astype(jnp.float32)


def _compute_positions(inp):
    bsz, seq = inp.shape
    return pl.pallas_call(
        _positions_body,
        out_shape=(
            jax.ShapeDtypeStruct((bsz, seq), jnp.int32),
            jax.ShapeDtypeStruct((bsz, seq), jnp.float32),
        ),
    )(inp)


def _make_sc_gather(num_tokens, d_model, chunk, nbuf):
    nchunks_total = num_tokens // chunk
    chunks_per_w = nchunks_total // _NW
    lookahead = nbuf - 1
    mesh = plsc.VectorSubcoreMesh(
        core_axis_name="c", subcore_axis_name="s", num_cores=_NC, num_subcores=_NS
    )

    @functools.partial(
        pl.kernel,
        mesh=mesh,
        compiler_params=pltpu.CompilerParams(needs_layout_passes=False),
        out_type=jax.ShapeDtypeStruct((num_tokens, d_model), jnp.float32),
        scratch_types=[
            pltpu.VMEM((chunks_per_w, chunk), jnp.int32),
            pltpu.VMEM((chunks_per_w, chunk), jnp.float32),
            [pltpu.VMEM((chunk, d_model), jnp.float32) for _ in range(nbuf)],
            [pltpu.SemaphoreType.DMA for _ in range(nbuf)],
            [pltpu.SemaphoreType.DMA for _ in range(nbuf)],
        ],
    )
    def sc_gather(table_hbm, pos_hbm, maskf_hbm, out_hbm, idx_v, mf_v, rows, gsem, ssem):
        wid = lax.axis_index("s") * _NC + lax.axis_index("c")
        rbase = wid * chunks_per_w
        tbase = rbase * chunk
        pltpu.sync_copy(pos_hbm.at[pl.ds(rbase, chunks_per_w)], idx_v)
        pltpu.sync_copy(maskf_hbm.at[pl.ds(rbase, chunks_per_w)], mf_v)

        def start_gather(c):
            b = c % nbuf
            return pltpu.async_copy(table_hbm.at[idx_v.at[c]], rows[b], gsem[b])

        gathers = {}
        scatters = {}
        for c in range(min(lookahead, chunks_per_w)):
            gathers[c] = start_gather(c)

        for c in range(chunks_per_w):
            b = c % nbuf
            cn = c + lookahead
            if cn < chunks_per_w:
                bn = cn % nbuf
                if cn >= nbuf:
                    scatters[cn - nbuf].wait()
                gathers[cn] = start_gather(cn)
            gathers[c].wait()

            def tok_body(t, carry):
                mvec = plsc.load_gather(
                    mf_v,
                    [jnp.full((_L,), c, jnp.int32), jnp.full((_L,), t, jnp.int32)],
                )
                for d in range(d_model // _L):
                    sl = (t, pl.ds(d * _L, _L))
                    rows[b][sl] = rows[b][sl] * mvec
                return carry

            lax.fori_loop(0, chunk, tok_body, 0)

            scatters[c] = pltpu.async_copy(
                rows[b], out_hbm.at[pl.ds(tbase + c * chunk, chunk)], ssem[b]
            )

        for c in range(max(0, chunks_per_w - nbuf), chunks_per_w):
            scatters[c].wait()

    return sc_gather


def kernel(weights, input):
    bsz, seq = input.shape
    num_tokens = bsz * seq
    d_model = weights.shape[1]
    chunk = 32

    pos, maskf = _compute_positions(input)
    pos = pos.reshape(num_tokens // chunk, chunk)
    maskf = maskf.reshape(num_tokens // chunk, chunk)

    gather = _make_sc_gather(num_tokens, d_model, chunk, nbuf=3)
    out = gather(weights, pos, maskf)
    return out.reshape(bsz, seq, d_model)
